# baseline (speedup 1.0000x reference)
import functools

import jax
import jax.numpy as jnp
from jax import lax
from jax.experimental import pallas as pl
from jax.experimental.pallas import tpu as pltpu
from jax.experimental.pallas import tpu_sc as plsc

N = 10000
E = 320000
H = 128
NORM_FACTOR = 100.0

NC = 2
NS = 16
NW = NC * NS
CH = 128
NCH = 80
PER_W = NCH * CH
EPAD = NW * PER_W
NB = 2
NBS = 2
XW = 16
NPAD = 10240
ROWS_PER_TILE = NPAD // NS

BE = 2048
BN = 1000


def _mesh():
    return plsc.VectorSubcoreMesh(
        core_axis_name="c", subcore_axis_name="s", num_cores=NC, num_subcores=NS
    )


def _silu(v):
    return v * (1.0 / (1.0 + jnp.exp(-v)))


def _rsqrt16(v):
    i = plsc.bitcast(v, jnp.int32)
    i = jnp.int32(0x5F3759DF) - lax.shift_right_arithmetic(i, 1)
    y = plsc.bitcast(i, jnp.float32)
    for _ in range(3):
        y = y * (1.5 - 0.5 * v * y * y)
    return y


def _sc_coord(xs, ys, zs, row, col):
    @functools.partial(
        pl.kernel,
        out_type=tuple(jax.ShapeDtypeStruct((EPAD,), jnp.float32) for _ in range(4)),
        mesh=_mesh(),
        scratch_types=(
            [pltpu.VMEM((N,), jnp.float32)] * 3
            + [pltpu.VMEM((PER_W,), jnp.int32)] * 2
            + [pltpu.VMEM((PER_W,), jnp.float32)] * 4
        ),
        compiler_params=pltpu.CompilerParams(needs_layout_passes=False),
    )
    def k(xs_hbm, ys_hbm, zs_hbm, row_hbm, col_hbm,
          rad_hbm, cdx_hbm, cdy_hbm, cdz_hbm,
          xt, yt, zt, idx1, idx2, radb, cdxb, cdyb, cdzb):
        wid = lax.axis_index("s") * NC + lax.axis_index("c")
        base = wid * PER_W
        pltpu.sync_copy(xs_hbm, xt)
        pltpu.sync_copy(ys_hbm, yt)
        pltpu.sync_copy(zs_hbm, zt)
        pltpu.sync_copy(row_hbm.at[pl.ds(base, PER_W)], idx1)
        pltpu.sync_copy(col_hbm.at[pl.ds(base, PER_W)], idx2)

        def group(g, _):
            sl = pl.ds(g * 16, 16)
            r = idx1[sl]
            c = idx2[sl]
            dx = plsc.load_gather(xt, [r]) - plsc.load_gather(xt, [c])
            dy = plsc.load_gather(yt, [r]) - plsc.load_gather(yt, [c])
            dz = plsc.load_gather(zt, [r]) - plsc.load_gather(zt, [c])
            rad = dx * dx + dy * dy + dz * dz
            y = _rsqrt16(rad + 1e-8)
            s = y / (1.0 + y)
            radb[sl] = rad
            cdxb[sl] = dx * s
            cdyb[sl] = dy * s
            cdzb[sl] = dz * s
            return 0

        lax.fori_loop(0, PER_W // 16, group, 0, unroll=2)
        pltpu.sync_copy(radb, rad_hbm.at[pl.ds(base, PER_W)])
        pltpu.sync_copy(cdxb, cdx_hbm.at[pl.ds(base, PER_W)])
        pltpu.sync_copy(cdyb, cdy_hbm.at[pl.ds(base, PER_W)])
        pltpu.sync_copy(cdzb, cdz_hbm.at[pl.ds(base, PER_W)])

    return k(xs, ys, zs, row, col)


def _sc_gather_add(g1, g2, row3d, col3d):
    @functools.partial(
        pl.kernel,
        out_type=jax.ShapeDtypeStruct((EPAD, H), jnp.float32),
        mesh=_mesh(),
        scratch_types=(
            [pltpu.VMEM((NCH, CH), jnp.int32)] * 2
            + [pltpu.VMEM((CH, H), jnp.float32)] * (2 * NB)
            + [pltpu.VMEM((CH, H), jnp.float32)] * 2
            + [pltpu.SemaphoreType.DMA] * (2 * NB + 2)
        ),
    )
    def k(g1_hbm, g2_hbm, row_hbm, col_hbm, out_hbm, idxr, idxc, *rest):
        bufs1 = rest[0:NB]
        bufs2 = rest[NB : 2 * NB]
        obufs = rest[2 * NB : 2 * NB + 2]
        sem1 = rest[2 * NB + 2 : 3 * NB + 2]
        sem2 = rest[3 * NB + 2 : 4 * NB + 2]
        wsem = rest[4 * NB + 2 : 4 * NB + 4]
        wid = lax.axis_index("s") * NC + lax.axis_index("c")
        base = wid * PER_W
        pltpu.sync_copy(row_hbm.at[wid], idxr)
        pltpu.sync_copy(col_hbm.at[wid], idxc)
        for b in range(NB):
            pltpu.async_copy(g1_hbm.at[idxr.at[b]], bufs1[b], sem1[b])
            pltpu.async_copy(g2_hbm.at[idxc.at[b]], bufs2[b], sem2[b])

        nrounds = NCH // (2 * NB)

        def body(j, _):
            for bb in range(2 * NB):
                b = bb % NB
                w = bb % 2
                i = j * 2 * NB + bb
                pltpu.make_async_copy(g1_hbm.at[idxr.at[i]], bufs1[b], sem1[b]).wait()
                pltpu.make_async_copy(g2_hbm.at[idxc.at[i]], bufs2[b], sem2[b]).wait()

                @pl.when(i > 1)
                def _():
                    pltpu.make_async_copy(
                        obufs[w], out_hbm.at[pl.ds(base, CH)], wsem[w]
                    ).wait()

                def addrow(r, _):
                    for l in range(H // 16):
                        sl = pl.ds(l * 16, 16)
                        obufs[w][r, sl] = bufs1[b][r, sl] + bufs2[b][r, sl]
                    return 0

                lax.fori_loop(0, CH, addrow, 0, unroll=2)

                @pl.when(i + NB < NCH)
                def _():
                    pltpu.async_copy(g1_hbm.at[idxr.at[i + NB]], bufs1[b], sem1[b])
                    pltpu.async_copy(g2_hbm.at[idxc.at[i + NB]], bufs2[b], sem2[b])

                pltpu.async_copy(obufs[w], out_hbm.at[pl.ds(base + i * CH, CH)], wsem[w])
            return 0

        lax.fori_loop(0, nrounds, body, 0, unroll=False)
        for w in range(2):
            pltpu.make_async_copy(obufs[w], out_hbm.at[pl.ds(base, CH)], wsem[w]).wait()

    return k(g1, g2, row3d, col3d)


def _sc_scatter_add(vals, row3d, zeros):
    @functools.partial(
        pl.kernel,
        out_type=jax.ShapeDtypeStruct((NC, NPAD, H), jnp.float32),
        mesh=_mesh(),
        scratch_types=(
            [pltpu.VMEM((NCH, CH), jnp.int32)]
            + [pltpu.VMEM((CH, H), jnp.float32)] * NBS
            + [pltpu.VMEM_SHARED((NPAD, H), jnp.float32)]
            + [pltpu.SemaphoreType.DMA] * NBS
        ),
    )
    def k(vals_hbm, row_hbm, zero_hbm, out_hbm, idx2d, *rest):
        bufs = rest[0:NBS]
        acc = rest[NBS]
        sems = rest[NBS + 1 : 2 * NBS + 1]
        c = lax.axis_index("c")
        s = lax.axis_index("s")
        wid = s * NC + c
        r0 = s * ROWS_PER_TILE
        base = wid * PER_W
        pltpu.sync_copy(
            zero_hbm.at[pl.ds(r0, ROWS_PER_TILE)], acc.at[pl.ds(r0, ROWS_PER_TILE)]
        )
        pltpu.sync_copy(row_hbm.at[wid], idx2d)
        for b in range(NBS):
            pltpu.async_copy(vals_hbm.at[pl.ds(base + b * CH, CH)], bufs[b], sems[b])
        plsc.subcore_barrier()

        nrounds = (NCH + NBS - 1) // NBS

        def body(j, _):
            for b in range(NBS):
                i = j * NBS + b

                @pl.when(i < NCH)
                def _():
                    pltpu.make_async_copy(
                        vals_hbm.at[pl.ds(base, CH)], bufs[b], sems[b]
                    ).wait()
                    pltpu.sync_copy(bufs[b], acc.at[idx2d.at[i]], add=True)

                    @pl.when(i + NBS < NCH)
                    def _():
                        pltpu.async_copy(
                            vals_hbm.at[pl.ds(base + (i + NBS) * CH, CH)],
                            bufs[b], sems[b],
                        )
            return 0

        lax.fori_loop(0, nrounds, body, 0, unroll=False)
        plsc.subcore_barrier()
        pltpu.sync_copy(
            acc.at[pl.ds(r0, ROWS_PER_TILE)], out_hbm.at[c, pl.ds(r0, ROWS_PER_TILE)]
        )

    return k(vals, row3d, zeros)


def _sc_scatter_equiv(phi, cdx, cdy, cdz, row3d, zeros, consts):
    @functools.partial(
        pl.kernel,
        out_type=jax.ShapeDtypeStruct((NC, NPAD, H), jnp.float32),
        mesh=_mesh(),
        scratch_types=(
            [pltpu.VMEM((NCH, CH), jnp.int32)]
            + [pltpu.VMEM((CH,), jnp.float32)] * 8
            + [pltpu.VMEM((CH, H), jnp.float32)]
            + [pltpu.VMEM((8, H), jnp.float32)]
            + [pltpu.VMEM_SHARED((NPAD, H), jnp.float32)]
            + [pltpu.SemaphoreType.DMA] * 2
        ),
    )
    def k(phi_hbm, cdx_hbm, cdy_hbm, cdz_hbm, row_hbm, zero_hbm, consts_hbm,
          out_hbm, idx2d, *rest):
        sbufs = (rest[0:4], rest[4:8])
        tbuf = rest[8]
        cv = rest[9]
        acc = rest[10]
        sems = rest[11:13]
        srcs = (phi_hbm, cdx_hbm, cdy_hbm, cdz_hbm)
        c = lax.axis_index("c")
        s = lax.axis_index("s")
        wid = s * NC + c
        r0 = s * ROWS_PER_TILE
        base = wid * PER_W
        pltpu.sync_copy(
            zero_hbm.at[pl.ds(r0, ROWS_PER_TILE)], acc.at[pl.ds(r0, ROWS_PER_TILE)]
        )
        pltpu.sync_copy(row_hbm.at[wid], idx2d)
        pltpu.sync_copy(consts_hbm, cv)
        for b in range(2):
            for q in range(4):
                pltpu.async_copy(
                    srcs[q].at[pl.ds(base + b * CH, CH)], sbufs[b][q], sems[b]
                )

        zv = cv[3, pl.ds(0, 16)]

        def zrow(r, _):
            for l in range(H // 16):
                tbuf[r, pl.ds(l * 16, 16)] = zv
            return 0

        lax.fori_loop(0, CH, zrow, 0, unroll=2)
        plsc.subcore_barrier()

        e0 = cv[0, pl.ds(0, 16)]
        e1 = cv[1, pl.ds(0, 16)]
        e2 = cv[2, pl.ds(0, 16)]

        def body(j, _):
            for b in range(2):
                i = j * 2 + b
                for q in range(4):
                    pltpu.make_async_copy(
                        srcs[q].at[pl.ds(base, CH)], sbufs[b][q], sems[b]
                    ).wait()

                def tgrp(g, _):
                    sl = pl.ds(g * 16, 16)
                    p16 = sbufs[b][0][sl]
                    sx16 = sbufs[b][1][sl]
                    sy16 = sbufs[b][2][sl]
                    sz16 = sbufs[b][3][sl]
                    for rr in range(16):
                        cd = sx16[rr] * e0 + sy16[rr] * e1 + sz16[rr] * e2
                        tbuf[g * 16 + rr, pl.ds(0, 16)] = cd * p16[rr]
                    return 0

                lax.fori_loop(0, CH // 16, tgrp, 0, unroll=False)
                pltpu.sync_copy(tbuf, acc.at[idx2d.at[i]], add=True)

                @pl.when(i + 2 < NCH)
                def _():
                    for q in range(4):
                        pltpu.async_copy(
                            srcs[q].at[pl.ds(base + (i + 2) * CH, CH)],
                            sbufs[b][q], sems[b],
                        )
            return 0

        lax.fori_loop(0, NCH // 2, body, 0, unroll=False)
        plsc.subcore_barrier()
        pltpu.sync_copy(
            acc.at[pl.ds(r0, ROWS_PER_TILE)], out_hbm.at[c, pl.ds(r0, ROWS_PER_TILE)]
        )

    return k(phi, cdx, cdy, cdz, row3d, zeros, consts)


def _tc_project(hh, wcat):
    def body(h_ref, w_ref, g1_ref, g2_ref):
        g = jnp.dot(h_ref[...], w_ref[...], preferred_element_type=jnp.float32)
        g1_ref[...] = g[:, :H]
        g2_ref[...] = g[:, H:]

    return pl.pallas_call(
        body,
        grid=(N // BN,),
        in_specs=[
            pl.BlockSpec((BN, H), lambda i: (i, 0)),
            pl.BlockSpec((H, 2 * H), lambda i: (0, 0)),
        ],
        out_specs=[pl.BlockSpec((BN, H), lambda i: (i, 0))] * 2,
        out_shape=[jax.ShapeDtypeStruct((N, H), jnp.float32)] * 2,
    )(hh, wcat)


def _tc_edge_mlp(s, scal2, w2, smalls):
    def body(s_ref, sc_ref, w2_ref, sm_ref, out_ref):
        i = pl.program_id(0)
        term = lax.dot_general(
            sc_ref[...], sm_ref[1:3, :],
            dimension_numbers=(((0,), (0,)), ((), ())),
            preferred_element_type=jnp.float32,
        )
        m1 = _silu(s_ref[...] + term + sm_ref[0:1, :])
        m = jnp.dot(m1, w2_ref[...], preferred_element_type=jnp.float32) + sm_ref[3:4, :]
        m = _silu(m)
        att = jnp.sum(m * sm_ref[4:5, :], axis=1, keepdims=True) + sm_ref[5:6, 0:1]
        ef = m * (1.0 / (1.0 + jnp.exp(-att)))
        eid = i * BE + lax.broadcasted_iota(jnp.int32, (BE, 1), 0)
        out_ref[...] = jnp.where(eid < E, ef, 0.0)

    return pl.pallas_call(
        body,
        grid=(EPAD // BE,),
        in_specs=[
            pl.BlockSpec((BE, H), lambda i: (i, 0)),
            pl.BlockSpec((2, BE), lambda i: (0, i)),
            pl.BlockSpec((H, H), lambda i: (0, 0)),
            pl.BlockSpec((8, H), lambda i: (0, 0)),
        ],
        out_specs=pl.BlockSpec((BE, H), lambda i: (i, 0)),
        out_shape=jax.ShapeDtypeStruct((EPAD, H), jnp.float32),
    )(s, scal2, w2, smalls)


def _tc_edge_equiv(s, scal2, w2, smalls):
    def body(s_ref, sc_ref, w2_ref, sm_ref, out_ref):
        i = pl.program_id(0)
        term = lax.dot_general(
            sc_ref[...], sm_ref[1:3, :],
            dimension_numbers=(((0,), (0,)), ((), ())),
            preferred_element_type=jnp.float32,
        )
        m1 = _silu(s_ref[...] + term + sm_ref[0:1, :])
        cfeat = jnp.dot(m1, w2_ref[...], preferred_element_type=jnp.float32) + sm_ref[3:4, :]
        cfeat = _silu(cfeat)
        phi_t = lax.dot_general(
            sm_ref[4:5, :], cfeat,
            dimension_numbers=(((1,), (1,)), ((), ())),
            preferred_element_type=jnp.float32,
        )
        eid = i * BE + lax.broadcasted_iota(jnp.int32, (1, BE), 1)
        out_ref[...] = jnp.where(eid < E, phi_t, 0.0)[None]

    return pl.pallas_call(
        body,
        grid=(EPAD // BE,),
        in_specs=[
            pl.BlockSpec((BE, H), lambda i: (i, 0)),
            pl.BlockSpec((2, BE), lambda i: (0, i)),
            pl.BlockSpec((H, H), lambda i: (0, 0)),
            pl.BlockSpec((8, H), lambda i: (0, 0)),
        ],
        out_specs=pl.BlockSpec((1, 1, BE), lambda i: (i, 0, 0)),
        out_shape=jax.ShapeDtypeStruct((EPAD // BE, 1, BE), jnp.float32),
    )(s, scal2, w2, smalls)


def _tc_node(hh, parts, n1, n2, smalls):
    def body(h_ref, pa_ref, pb_ref, n1_ref, n2_ref, sm_ref, out_ref):
        hcur = h_ref[...]
        agg = (pa_ref[0] + pb_ref[0]) * (1.0 / NORM_FACTOR)
        u = (
            jnp.dot(hcur, n1_ref[:H, :], preferred_element_type=jnp.float32)
            + jnp.dot(agg, n1_ref[H:, :], preferred_element_type=jnp.float32)
            + sm_ref[0:1, :]
        )
        u = _silu(u)
        u = jnp.dot(u, n2_ref[...], preferred_element_type=jnp.float32) + sm_ref[1:2, :]
        out_ref[...] = hcur + u

    return pl.pallas_call(
        body,
        grid=(N // BN,),
        in_specs=[
            pl.BlockSpec((BN, H), lambda i: (i, 0)),
            pl.BlockSpec((1, BN, H), lambda i: (0, i, 0)),
            pl.BlockSpec((1, BN, H), lambda i: (1, i, 0)),
            pl.BlockSpec((2 * H, H), lambda i: (0, 0)),
            pl.BlockSpec((H, H), lambda i: (0, 0)),
            pl.BlockSpec((8, H), lambda i: (0, 0)),
        ],
        out_specs=pl.BlockSpec((BN, H), lambda i: (i, 0)),
        out_shape=jax.ShapeDtypeStruct((N, H), jnp.float32),
    )(hh, parts, parts, n1, n2, smalls)


def _tc_final_x(xp, parts):
    def body(x_ref, pa_ref, pb_ref, out_ref):
        out_ref[...] = x_ref[...] + (pa_ref[0] + pb_ref[0]) * (1.0 / NORM_FACTOR)

    return pl.pallas_call(
        body,
        grid=(N // BN,),
        in_specs=[
            pl.BlockSpec((BN, H), lambda i: (i, 0)),
            pl.BlockSpec((1, BN, H), lambda i: (0, i, 0)),
            pl.BlockSpec((1, BN, H), lambda i: (1, i, 0)),
        ],
        out_specs=pl.BlockSpec((BN, H), lambda i: (i, 0)),
        out_shape=jax.ShapeDtypeStruct((N, H), jnp.float32),
    )(xp, parts, parts)


def _pack_gcl_smalls(p):
    z = jnp.zeros((8, H), jnp.float32)
    z = z.at[0].set(p["e1"]["b"])
    z = z.at[1].set(p["e1"]["w"][2 * H])
    z = z.at[2].set(p["e1"]["w"][2 * H + 1])
    z = z.at[3].set(p["e2"]["b"])
    z = z.at[4].set(p["att"]["w"][:, 0])
    z = z.at[5].set(jnp.full((H,), p["att"]["b"][0]))
    return z


def _pack_equiv_smalls(p):
    z = jnp.zeros((8, H), jnp.float32)
    z = z.at[0].set(p["c1"]["b"])
    z = z.at[1].set(p["c1"]["w"][2 * H])
    z = z.at[2].set(p["c1"]["w"][2 * H + 1])
    z = z.at[3].set(p["c2"]["b"])
    z = z.at[4].set(p["c3w"][:, 0])
    return z


def kernel(h, x, edge_index, edge_attr, params):
    row = jnp.pad(edge_index[0], (0, EPAD - E))
    col = jnp.pad(edge_index[1], (0, EPAD - E))
    ea = jnp.pad(edge_attr[:, 0], (0, EPAD - E))
    row3d = row.reshape(NW, NCH, CH)
    col3d = col.reshape(NW, NCH, CH)
    zeros_h = jnp.zeros((NPAD, H), jnp.float32)

    rad, cdx, cdy, cdz = _sc_coord(
        jnp.asarray(x[:, 0]), jnp.asarray(x[:, 1]), jnp.asarray(x[:, 2]), row, col
    )
    scal2 = jnp.stack([rad, ea])

    hh = h
    for i in range(2):
        p = params["gcl%d" % i]
        w1 = p["e1"]["w"]
        wcat = jnp.concatenate([w1[:H], w1[H : 2 * H]], axis=1)
        g1, g2 = _tc_project(hh, wcat)
        s = _sc_gather_add(g1, g2, row3d, col3d)
        ef = _tc_edge_mlp(s, scal2, p["e2"]["w"], _pack_gcl_smalls(p))
        parts = _sc_scatter_add(ef, row3d, zeros_h)
        hh = _tc_node(hh, parts, p["n1"]["w"], p["n2"]["w"],
                      jnp.stack([p["n1"]["b"], p["n2"]["b"]] + [jnp.zeros((H,))] * 6))

    pe = params["equiv"]
    c1 = pe["c1"]["w"]
    wcat = jnp.concatenate([c1[:H], c1[H : 2 * H]], axis=1)
    g1, g2 = _tc_project(hh, wcat)
    s = _sc_gather_add(g1, g2, row3d, col3d)
    phi = _tc_edge_equiv(s, scal2, pe["c2"]["w"], _pack_equiv_smalls(pe)).reshape(EPAD)
    consts = (jnp.zeros((8, H), jnp.float32)
              .at[0, 0].set(1.0).at[1, 1].set(1.0).at[2, 2].set(1.0))
    parts = _sc_scatter_equiv(phi, cdx, cdy, cdz, row3d, zeros_h, consts)
    xp = jnp.concatenate([x, jnp.zeros((N, H - 3), jnp.float32)], axis=1)
    x16 = _tc_final_x(xp, parts)
    xx = x16[:, :3]
    return hh, xx


# speedup vs baseline: 1.1085x; 1.1085x over previous
import functools

import jax
import jax.numpy as jnp
from jax import lax
from jax.experimental import pallas as pl
from jax.experimental.pallas import tpu as pltpu
from jax.experimental.pallas import tpu_sc as plsc

N = 10000
E = 320000
H = 128
NORM_FACTOR = 100.0

NC = 2
NS = 16
NW = NC * NS
CH = 128
NCH = 80
PER_W = NCH * CH
EPAD = NW * PER_W
NB = 2
NBS = 2
XW = 16
NPAD = 10240
ROWS_PER_TILE = NPAD // NS

EH = EPAD // 2
NCHH = NCH // 2
PER_WH = NCHH * CH
BE = 2048
BN = 1000


def _mesh():
    return plsc.VectorSubcoreMesh(
        core_axis_name="c", subcore_axis_name="s", num_cores=NC, num_subcores=NS
    )


def _silu(v):
    return v * (1.0 / (1.0 + jnp.exp(-v)))


def _rsqrt16(v):
    i = plsc.bitcast(v, jnp.int32)
    i = jnp.int32(0x5F3759DF) - lax.shift_right_arithmetic(i, 1)
    y = plsc.bitcast(i, jnp.float32)
    for _ in range(3):
        y = y * (1.5 - 0.5 * v * y * y)
    return y


def _sc_coord(xs, ys, zs, row, col):
    @functools.partial(
        pl.kernel,
        out_type=tuple(jax.ShapeDtypeStruct((EPAD,), jnp.float32) for _ in range(4)),
        mesh=_mesh(),
        scratch_types=(
            [pltpu.VMEM((N,), jnp.float32)] * 3
            + [pltpu.VMEM((PER_W,), jnp.int32)] * 2
            + [pltpu.VMEM((PER_W,), jnp.float32)] * 4
        ),
        compiler_params=pltpu.CompilerParams(needs_layout_passes=False),
    )
    def k(xs_hbm, ys_hbm, zs_hbm, row_hbm, col_hbm,
          rad_hbm, cdx_hbm, cdy_hbm, cdz_hbm,
          xt, yt, zt, idx1, idx2, radb, cdxb, cdyb, cdzb):
        wid = lax.axis_index("s") * NC + lax.axis_index("c")
        base = wid * PER_W
        pltpu.sync_copy(xs_hbm, xt)
        pltpu.sync_copy(ys_hbm, yt)
        pltpu.sync_copy(zs_hbm, zt)
        pltpu.sync_copy(row_hbm.at[pl.ds(base, PER_W)], idx1)
        pltpu.sync_copy(col_hbm.at[pl.ds(base, PER_W)], idx2)

        def group(g, _):
            sl = pl.ds(g * 16, 16)
            r = idx1[sl]
            c = idx2[sl]
            dx = plsc.load_gather(xt, [r]) - plsc.load_gather(xt, [c])
            dy = plsc.load_gather(yt, [r]) - plsc.load_gather(yt, [c])
            dz = plsc.load_gather(zt, [r]) - plsc.load_gather(zt, [c])
            rad = dx * dx + dy * dy + dz * dz
            y = _rsqrt16(rad + 1e-8)
            s = y / (1.0 + y)
            radb[sl] = rad
            cdxb[sl] = dx * s
            cdyb[sl] = dy * s
            cdzb[sl] = dz * s
            return 0

        lax.fori_loop(0, PER_W // 16, group, 0, unroll=2)
        pltpu.sync_copy(radb, rad_hbm.at[pl.ds(base, PER_W)])
        pltpu.sync_copy(cdxb, cdx_hbm.at[pl.ds(base, PER_W)])
        pltpu.sync_copy(cdyb, cdy_hbm.at[pl.ds(base, PER_W)])
        pltpu.sync_copy(cdzb, cdz_hbm.at[pl.ds(base, PER_W)])

    return k(xs, ys, zs, row, col)


def _sc_gather_add(g1, g2, row3d, col3d):
    @functools.partial(
        pl.kernel,
        out_type=jax.ShapeDtypeStruct((EH, H), jnp.float32),
        mesh=_mesh(),
        scratch_types=(
            [pltpu.VMEM((NCHH, CH), jnp.int32)] * 2
            + [pltpu.VMEM((CH, H), jnp.float32)] * (2 * NB)
            + [pltpu.VMEM((CH, H), jnp.float32)] * 2
            + [pltpu.SemaphoreType.DMA] * (2 * NB + 2)
        ),
    )
    def k(g1_hbm, g2_hbm, row_hbm, col_hbm, out_hbm, idxr, idxc, *rest):
        bufs1 = rest[0:NB]
        bufs2 = rest[NB : 2 * NB]
        obufs = rest[2 * NB : 2 * NB + 2]
        sem1 = rest[2 * NB + 2 : 3 * NB + 2]
        sem2 = rest[3 * NB + 2 : 4 * NB + 2]
        wsem = rest[4 * NB + 2 : 4 * NB + 4]
        wid = lax.axis_index("s") * NC + lax.axis_index("c")
        base = wid * PER_WH
        pltpu.sync_copy(row_hbm.at[wid], idxr)
        pltpu.sync_copy(col_hbm.at[wid], idxc)
        for b in range(NB):
            pltpu.async_copy(g1_hbm.at[idxr.at[b]], bufs1[b], sem1[b])
            pltpu.async_copy(g2_hbm.at[idxc.at[b]], bufs2[b], sem2[b])

        nrounds = NCHH // (2 * NB)

        def body(j, _):
            for bb in range(2 * NB):
                b = bb % NB
                w = bb % 2
                i = j * 2 * NB + bb
                pltpu.make_async_copy(g1_hbm.at[idxr.at[i]], bufs1[b], sem1[b]).wait()
                pltpu.make_async_copy(g2_hbm.at[idxc.at[i]], bufs2[b], sem2[b]).wait()

                @pl.when(i > 1)
                def _():
                    pltpu.make_async_copy(
                        obufs[w], out_hbm.at[pl.ds(base, CH)], wsem[w]
                    ).wait()

                def addrow(r, _):
                    for l in range(H // 16):
                        sl = pl.ds(l * 16, 16)
                        obufs[w][r, sl] = bufs1[b][r, sl] + bufs2[b][r, sl]
                    return 0

                lax.fori_loop(0, CH, addrow, 0, unroll=2)

                @pl.when(i + NB < NCHH)
                def _():
                    pltpu.async_copy(g1_hbm.at[idxr.at[i + NB]], bufs1[b], sem1[b])
                    pltpu.async_copy(g2_hbm.at[idxc.at[i + NB]], bufs2[b], sem2[b])

                pltpu.async_copy(obufs[w], out_hbm.at[pl.ds(base + i * CH, CH)], wsem[w])
            return 0

        lax.fori_loop(0, nrounds, body, 0, unroll=False)
        for w in range(2):
            pltpu.make_async_copy(obufs[w], out_hbm.at[pl.ds(base, CH)], wsem[w]).wait()

    return k(g1, g2, row3d, col3d)


def _sc_scatter_add(vals, row3d, zeros):
    @functools.partial(
        pl.kernel,
        out_type=jax.ShapeDtypeStruct((NC, NPAD, H), jnp.float32),
        mesh=_mesh(),
        scratch_types=(
            [pltpu.VMEM((NCHH, CH), jnp.int32)]
            + [pltpu.VMEM((CH, H), jnp.float32)] * NBS
            + [pltpu.VMEM_SHARED((NPAD, H), jnp.float32)]
            + [pltpu.SemaphoreType.DMA] * NBS
        ),
    )
    def k(vals_hbm, row_hbm, zero_hbm, out_hbm, idx2d, *rest):
        bufs = rest[0:NBS]
        acc = rest[NBS]
        sems = rest[NBS + 1 : 2 * NBS + 1]
        c = lax.axis_index("c")
        s = lax.axis_index("s")
        wid = s * NC + c
        r0 = s * ROWS_PER_TILE
        base = wid * PER_WH
        pltpu.sync_copy(
            zero_hbm.at[pl.ds(r0, ROWS_PER_TILE)], acc.at[pl.ds(r0, ROWS_PER_TILE)]
        )
        pltpu.sync_copy(row_hbm.at[wid], idx2d)
        for b in range(NBS):
            pltpu.async_copy(vals_hbm.at[pl.ds(base + b * CH, CH)], bufs[b], sems[b])
        plsc.subcore_barrier()

        nrounds = (NCHH + NBS - 1) // NBS

        def body(j, _):
            for b in range(NBS):
                i = j * NBS + b

                @pl.when(i < NCHH)
                def _():
                    pltpu.make_async_copy(
                        vals_hbm.at[pl.ds(base, CH)], bufs[b], sems[b]
                    ).wait()
                    pltpu.sync_copy(bufs[b], acc.at[idx2d.at[i]], add=True)

                    @pl.when(i + NBS < NCHH)
                    def _():
                        pltpu.async_copy(
                            vals_hbm.at[pl.ds(base + (i + NBS) * CH, CH)],
                            bufs[b], sems[b],
                        )
            return 0

        lax.fori_loop(0, nrounds, body, 0, unroll=False)
        plsc.subcore_barrier()
        pltpu.sync_copy(
            acc.at[pl.ds(r0, ROWS_PER_TILE)], out_hbm.at[c, pl.ds(r0, ROWS_PER_TILE)]
        )

    return k(vals, row3d, zeros)


def _sc_scatter_equiv(phi, cdx, cdy, cdz, row3d, zeros, consts, ebase):
    @functools.partial(
        pl.kernel,
        out_type=jax.ShapeDtypeStruct((NC, NPAD, H), jnp.float32),
        mesh=_mesh(),
        scratch_types=(
            [pltpu.VMEM((NCHH, CH), jnp.int32)]
            + [pltpu.VMEM((CH,), jnp.float32)] * 8
            + [pltpu.VMEM((CH, H), jnp.float32)]
            + [pltpu.VMEM((8, H), jnp.float32)]
            + [pltpu.VMEM_SHARED((NPAD, H), jnp.float32)]
            + [pltpu.SemaphoreType.DMA] * 2
        ),
    )
    def k(phi_hbm, cdx_hbm, cdy_hbm, cdz_hbm, row_hbm, zero_hbm, consts_hbm,
          out_hbm, idx2d, *rest):
        sbufs = (rest[0:4], rest[4:8])
        tbuf = rest[8]
        cv = rest[9]
        acc = rest[10]
        sems = rest[11:13]
        srcs = (phi_hbm, cdx_hbm, cdy_hbm, cdz_hbm)
        c = lax.axis_index("c")
        s = lax.axis_index("s")
        wid = s * NC + c
        r0 = s * ROWS_PER_TILE
        base = wid * PER_WH
        cbase = ebase + base
        pltpu.sync_copy(
            zero_hbm.at[pl.ds(r0, ROWS_PER_TILE)], acc.at[pl.ds(r0, ROWS_PER_TILE)]
        )
        pltpu.sync_copy(row_hbm.at[wid], idx2d)
        pltpu.sync_copy(consts_hbm, cv)
        for b in range(2):
            for q in range(4):
                pltpu.async_copy(
                    srcs[q].at[pl.ds((cbase if q else base) + b * CH, CH)], sbufs[b][q], sems[b]
                )

        zv = cv[3, pl.ds(0, 16)]

        def zrow(r, _):
            for l in range(H // 16):
                tbuf[r, pl.ds(l * 16, 16)] = zv
            return 0

        lax.fori_loop(0, CH, zrow, 0, unroll=2)
        plsc.subcore_barrier()

        e0 = cv[0, pl.ds(0, 16)]
        e1 = cv[1, pl.ds(0, 16)]
        e2 = cv[2, pl.ds(0, 16)]

        def body(j, _):
            for b in range(2):
                i = j * 2 + b
                for q in range(4):
                    pltpu.make_async_copy(
                        srcs[q].at[pl.ds((cbase if q else base), CH)], sbufs[b][q], sems[b]
                    ).wait()

                def tgrp(g, _):
                    sl = pl.ds(g * 16, 16)
                    p16 = sbufs[b][0][sl]
                    sx16 = sbufs[b][1][sl]
                    sy16 = sbufs[b][2][sl]
                    sz16 = sbufs[b][3][sl]
                    for rr in range(16):
                        cd = sx16[rr] * e0 + sy16[rr] * e1 + sz16[rr] * e2
                        tbuf[g * 16 + rr, pl.ds(0, 16)] = cd * p16[rr]
                    return 0

                lax.fori_loop(0, CH // 16, tgrp, 0, unroll=False)
                pltpu.sync_copy(tbuf, acc.at[idx2d.at[i]], add=True)

                @pl.when(i + 2 < NCHH)
                def _():
                    for q in range(4):
                        pltpu.async_copy(
                            srcs[q].at[pl.ds((cbase if q else base) + (i + 2) * CH, CH)],
                            sbufs[b][q], sems[b],
                        )
            return 0

        lax.fori_loop(0, NCHH // 2, body, 0, unroll=False)
        plsc.subcore_barrier()
        pltpu.sync_copy(
            acc.at[pl.ds(r0, ROWS_PER_TILE)], out_hbm.at[c, pl.ds(r0, ROWS_PER_TILE)]
        )

    return k(phi, cdx, cdy, cdz, row3d, zeros, consts)


def _tc_project(hh, wcat):
    def body(h_ref, w_ref, g1_ref, g2_ref):
        g = jnp.dot(h_ref[...], w_ref[...], preferred_element_type=jnp.float32)
        g1_ref[...] = g[:, :H]
        g2_ref[...] = g[:, H:]

    return pl.pallas_call(
        body,
        grid=(N // BN,),
        in_specs=[
            pl.BlockSpec((BN, H), lambda i: (i, 0)),
            pl.BlockSpec((H, 2 * H), lambda i: (0, 0)),
        ],
        out_specs=[pl.BlockSpec((BN, H), lambda i: (i, 0))] * 2,
        out_shape=[jax.ShapeDtypeStruct((N, H), jnp.float32)] * 2,
    )(hh, wcat)


def _tc_edge_mlp(s, scal2, w2, smalls, ebase):
    def body(s_ref, sc_ref, w2_ref, sm_ref, out_ref):
        i = pl.program_id(0)
        term = lax.dot_general(
            sc_ref[...], sm_ref[1:3, :],
            dimension_numbers=(((0,), (0,)), ((), ())),
            preferred_element_type=jnp.float32,
        )
        m1 = _silu(s_ref[...] + term + sm_ref[0:1, :])
        m = jnp.dot(m1, w2_ref[...], preferred_element_type=jnp.float32) + sm_ref[3:4, :]
        m = _silu(m)
        att = jnp.sum(m * sm_ref[4:5, :], axis=1, keepdims=True) + sm_ref[5:6, 0:1]
        ef = m * (1.0 / (1.0 + jnp.exp(-att)))
        eid = ebase + i * BE + lax.broadcasted_iota(jnp.int32, (BE, 1), 0)
        out_ref[...] = jnp.where(eid < E, ef, 0.0)

    return pl.pallas_call(
        body,
        grid=(EH // BE,),
        in_specs=[
            pl.BlockSpec((BE, H), lambda i: (i, 0)),
            pl.BlockSpec((2, BE), lambda i, _e=ebase // BE: (0, i + _e)),
            pl.BlockSpec((H, H), lambda i: (0, 0)),
            pl.BlockSpec((8, H), lambda i: (0, 0)),
        ],
        out_specs=pl.BlockSpec((BE, H), lambda i: (i, 0)),
        out_shape=jax.ShapeDtypeStruct((EH, H), jnp.float32),
    )(s, scal2, w2, smalls)


def _tc_edge_equiv(s, scal2, w2, smalls, ebase):
    def body(s_ref, sc_ref, w2_ref, sm_ref, out_ref):
        i = pl.program_id(0)
        term = lax.dot_general(
            sc_ref[...], sm_ref[1:3, :],
            dimension_numbers=(((0,), (0,)), ((), ())),
            preferred_element_type=jnp.float32,
        )
        m1 = _silu(s_ref[...] + term + sm_ref[0:1, :])
        cfeat = jnp.dot(m1, w2_ref[...], preferred_element_type=jnp.float32) + sm_ref[3:4, :]
        cfeat = _silu(cfeat)
        phi_t = lax.dot_general(
            sm_ref[4:5, :], cfeat,
            dimension_numbers=(((1,), (1,)), ((), ())),
            preferred_element_type=jnp.float32,
        )
        eid = ebase + i * BE + lax.broadcasted_iota(jnp.int32, (1, BE), 1)
        out_ref[...] = jnp.where(eid < E, phi_t, 0.0)[None]

    return pl.pallas_call(
        body,
        grid=(EH // BE,),
        in_specs=[
            pl.BlockSpec((BE, H), lambda i: (i, 0)),
            pl.BlockSpec((2, BE), lambda i, _e=ebase // BE: (0, i + _e)),
            pl.BlockSpec((H, H), lambda i: (0, 0)),
            pl.BlockSpec((8, H), lambda i: (0, 0)),
        ],
        out_specs=pl.BlockSpec((1, 1, BE), lambda i: (i, 0, 0)),
        out_shape=jax.ShapeDtypeStruct((EH // BE, 1, BE), jnp.float32),
    )(s, scal2, w2, smalls)


def _tc_node(hh, partsA, partsB, n1, n2, smalls):
    def body(h_ref, pa_ref, pb_ref, pc_ref, pd_ref, n1_ref, n2_ref, sm_ref, out_ref):
        hcur = h_ref[...]
        agg = (pa_ref[0] + pb_ref[0] + pc_ref[0] + pd_ref[0]) * (1.0 / NORM_FACTOR)
        u = (
            jnp.dot(hcur, n1_ref[:H, :], preferred_element_type=jnp.float32)
            + jnp.dot(agg, n1_ref[H:, :], preferred_element_type=jnp.float32)
            + sm_ref[0:1, :]
        )
        u = _silu(u)
        u = jnp.dot(u, n2_ref[...], preferred_element_type=jnp.float32) + sm_ref[1:2, :]
        out_ref[...] = hcur + u

    return pl.pallas_call(
        body,
        grid=(N // BN,),
        in_specs=[
            pl.BlockSpec((BN, H), lambda i: (i, 0)),
            pl.BlockSpec((1, BN, H), lambda i: (0, i, 0)),
            pl.BlockSpec((1, BN, H), lambda i: (1, i, 0)),
            pl.BlockSpec((1, BN, H), lambda i: (0, i, 0)),
            pl.BlockSpec((1, BN, H), lambda i: (1, i, 0)),
            pl.BlockSpec((2 * H, H), lambda i: (0, 0)),
            pl.BlockSpec((H, H), lambda i: (0, 0)),
            pl.BlockSpec((8, H), lambda i: (0, 0)),
        ],
        out_specs=pl.BlockSpec((BN, H), lambda i: (i, 0)),
        out_shape=jax.ShapeDtypeStruct((N, H), jnp.float32),
    )(hh, partsA, partsA, partsB, partsB, n1, n2, smalls)


def _tc_final_x(xp, partsA, partsB):
    def body(x_ref, pa_ref, pb_ref, pc_ref, pd_ref, out_ref):
        out_ref[...] = x_ref[...] + (
            pa_ref[0] + pb_ref[0] + pc_ref[0] + pd_ref[0]
        ) * (1.0 / NORM_FACTOR)

    return pl.pallas_call(
        body,
        grid=(N // BN,),
        in_specs=[
            pl.BlockSpec((BN, H), lambda i: (i, 0)),
            pl.BlockSpec((1, BN, H), lambda i: (0, i, 0)),
            pl.BlockSpec((1, BN, H), lambda i: (1, i, 0)),
            pl.BlockSpec((1, BN, H), lambda i: (0, i, 0)),
            pl.BlockSpec((1, BN, H), lambda i: (1, i, 0)),
        ],
        out_specs=pl.BlockSpec((BN, H), lambda i: (i, 0)),
        out_shape=jax.ShapeDtypeStruct((N, H), jnp.float32),
    )(xp, partsA, partsA, partsB, partsB)


def _pack_gcl_smalls(p):
    z = jnp.zeros((8, H), jnp.float32)
    z = z.at[0].set(p["e1"]["b"])
    z = z.at[1].set(p["e1"]["w"][2 * H])
    z = z.at[2].set(p["e1"]["w"][2 * H + 1])
    z = z.at[3].set(p["e2"]["b"])
    z = z.at[4].set(p["att"]["w"][:, 0])
    z = z.at[5].set(jnp.full((H,), p["att"]["b"][0]))
    return z


def _pack_equiv_smalls(p):
    z = jnp.zeros((8, H), jnp.float32)
    z = z.at[0].set(p["c1"]["b"])
    z = z.at[1].set(p["c1"]["w"][2 * H])
    z = z.at[2].set(p["c1"]["w"][2 * H + 1])
    z = z.at[3].set(p["c2"]["b"])
    z = z.at[4].set(p["c3w"][:, 0])
    return z


def kernel(h, x, edge_index, edge_attr, params):
    row = jnp.pad(edge_index[0], (0, EPAD - E))
    col = jnp.pad(edge_index[1], (0, EPAD - E))
    ea = jnp.pad(edge_attr[:, 0], (0, EPAD - E))
    rowA = row[:EH].reshape(NW, NCHH, CH)
    colA = col[:EH].reshape(NW, NCHH, CH)
    rowB = row[EH:].reshape(NW, NCHH, CH)
    colB = col[EH:].reshape(NW, NCHH, CH)
    zeros_h = jnp.zeros((NPAD, H), jnp.float32)

    rad, cdx, cdy, cdz = _sc_coord(
        jnp.asarray(x[:, 0]), jnp.asarray(x[:, 1]), jnp.asarray(x[:, 2]), row, col
    )
    scal2 = jnp.stack([rad, ea])

    hh = h
    for i in range(2):
        p = params["gcl%d" % i]
        w1 = p["e1"]["w"]
        wcat = jnp.concatenate([w1[:H], w1[H : 2 * H]], axis=1)
        g1, g2 = _tc_project(hh, wcat)
        sA = _sc_gather_add(g1, g2, rowA, colA)
        sB = _sc_gather_add(g1, g2, rowB, colB)
        sm = _pack_gcl_smalls(p)
        efA = _tc_edge_mlp(sA, scal2, p["e2"]["w"], sm, 0)
        efB = _tc_edge_mlp(sB, scal2, p["e2"]["w"], sm, EH)
        pA = _sc_scatter_add(efA, rowA, zeros_h)
        pB = _sc_scatter_add(efB, rowB, zeros_h)
        hh = _tc_node(hh, pA, pB, p["n1"]["w"], p["n2"]["w"],
                      jnp.stack([p["n1"]["b"], p["n2"]["b"]] + [jnp.zeros((H,))] * 6))

    pe = params["equiv"]
    c1 = pe["c1"]["w"]
    wcat = jnp.concatenate([c1[:H], c1[H : 2 * H]], axis=1)
    g1, g2 = _tc_project(hh, wcat)
    sA = _sc_gather_add(g1, g2, rowA, colA)
    sB = _sc_gather_add(g1, g2, rowB, colB)
    sm = _pack_equiv_smalls(pe)
    phiA = _tc_edge_equiv(sA, scal2, pe["c2"]["w"], sm, 0).reshape(EH)
    phiB = _tc_edge_equiv(sB, scal2, pe["c2"]["w"], sm, EH).reshape(EH)
    consts = (jnp.zeros((8, H), jnp.float32)
              .at[0, 0].set(1.0).at[1, 1].set(1.0).at[2, 2].set(1.0))
    pA = _sc_scatter_equiv(phiA, cdx, cdy, cdz, rowA, zeros_h, consts, 0)
    pB = _sc_scatter_equiv(phiB, cdx, cdy, cdz, rowB, zeros_h, consts, EH)
    xp = jnp.concatenate([x, jnp.zeros((N, H - 3), jnp.float32)], axis=1)
    x16 = _tc_final_x(xp, pA, pB)
    xx = x16[:, :3]
    return hh, xx
